# baseline (speedup 1.0000x reference)
import functools

import jax
import jax.numpy as jnp
from jax import lax
from jax.experimental import pallas as pl
from jax.experimental.pallas import tpu as pltpu
from jax.experimental.pallas import tpu_sc as plsc

DIM = 64
STRIP = 128
S = 8
NC = 2
NS = 16
NW = NC * NS


def _emb_gather(x2d, table, out, idx_v, rows_v, sem):
    wid = lax.axis_index("s") * NC + lax.axis_index("c")
    n_strips = x2d.shape[0]
    per_w = n_strips // NW
    chunks = per_w // S
    base = wid * per_w

    def body(c, carry):
        row0 = base + c * S
        pltpu.sync_copy(x2d.at[pl.ds(row0, S)], idx_v)
        handles = []
        for i in range(S):
            handles.append(
                pltpu.async_copy(
                    table.at[idx_v.at[i]],
                    rows_v.at[pl.ds(i * STRIP, STRIP)],
                    sem,
                )
            )
        for h in handles:
            h.wait()
        pltpu.sync_copy(rows_v, out.at[pl.ds(row0 * STRIP, S * STRIP)])
        return carry

    lax.fori_loop(0, chunks, body, 0)


def kernel(x, weight):
    B0, B1 = x.shape
    n = B0 * B1
    xf = x.reshape(n).astype(jnp.int32)
    n_strips = n // STRIP
    x2d = xf.reshape(n_strips, STRIP)

    mesh = plsc.VectorSubcoreMesh(core_axis_name="c", subcore_axis_name="s")
    run = pl.kernel(
        _emb_gather,
        mesh=mesh,
        out_type=jax.ShapeDtypeStruct((n, DIM), jnp.float32),
        scratch_types=[
            pltpu.VMEM((S, STRIP), jnp.int32),
            pltpu.VMEM((S * STRIP, DIM), jnp.float32),
            pltpu.SemaphoreType.DMA,
        ],
        compiler_params=pltpu.CompilerParams(use_tc_tiling_on_sc=False),
    )
    outf = run(x2d, weight)
    return outf.reshape(B0, B1, DIM)


# speedup vs baseline: 1.0164x; 1.0164x over previous
import jax
import jax.numpy as jnp
from jax import lax
from jax.experimental import pallas as pl
from jax.experimental.pallas import tpu as pltpu
from jax.experimental.pallas import tpu_sc as plsc

DIM = 64
STRIP = 128
S = 5
NC = 2
NS = 16
NW = NC * NS
CHUNK = S * STRIP


def _emb_gather(x2d, table, out, idx_v, rows_v, gsem, wsem, isem):
    wid = lax.axis_index("s") * NC + lax.axis_index("c")
    n_strips = x2d.shape[0]
    per_w = n_strips // NW
    chunks = per_w // S
    pairs = chunks // 2
    base = wid * per_w

    def row0(c):
        return base + c * S

    def fire_gathers(b):
        for i in range(S):
            pltpu.async_copy(
                table.at[idx_v.at[b, i]],
                rows_v.at[b, pl.ds(i * STRIP, STRIP)],
                gsem,
            )

    def drain_gathers(b):
        for i in range(S):
            pltpu.make_async_copy(
                table.at[idx_v.at[b, i]],
                rows_v.at[b, pl.ds(i * STRIP, STRIP)],
                gsem,
            ).wait()

    def fire_writeback(c, b):
        pltpu.async_copy(
            rows_v.at[b], out.at[pl.ds(row0(c) * STRIP, CHUNK)], wsem
        )

    def drain_writeback(c, b):
        pltpu.make_async_copy(
            rows_v.at[b], out.at[pl.ds(row0(c) * STRIP, CHUNK)], wsem
        ).wait()

    def stage_idx(c, b):
        pltpu.async_copy(x2d.at[pl.ds(row0(c), S)], idx_v.at[b], isem)

    def drain_idx(c, b):
        pltpu.make_async_copy(
            x2d.at[pl.ds(row0(c), S)], idx_v.at[b], isem
        ).wait()

    stage_idx(0, 0)
    drain_idx(0, 0)
    stage_idx(1, 1)
    fire_gathers(0)

    def pair_body(g, carry):
        c0 = 2 * g

        @pl.when(g > 0)
        def _():
            drain_writeback(c0 - 1, 1)

        drain_idx(c0 + 1, 1)
        fire_gathers(1)
        drain_gathers(0)
        fire_writeback(c0, 0)

        @pl.when(g < pairs - 1)
        def _():
            stage_idx(c0 + 2, 0)

        c1 = 2 * g + 1
        drain_writeback(c1 - 1, 0)

        @pl.when(g < pairs - 1)
        def _():
            drain_idx(c1 + 1, 0)
            fire_gathers(0)

        drain_gathers(1)
        fire_writeback(c1, 1)

        @pl.when(g < pairs - 1)
        def _():
            stage_idx(c1 + 2, 1)

        return carry

    lax.fori_loop(0, pairs, pair_body, 0)
    drain_writeback(chunks - 1, 1)


def kernel(x, weight):
    B0, B1 = x.shape
    n = B0 * B1
    xf = x.reshape(n).astype(jnp.int32)
    n_strips = n // STRIP
    x2d = xf.reshape(n_strips, STRIP)

    mesh = plsc.VectorSubcoreMesh(core_axis_name="c", subcore_axis_name="s")
    run = pl.kernel(
        _emb_gather,
        mesh=mesh,
        out_type=jax.ShapeDtypeStruct((n, DIM), jnp.float32),
        scratch_types=[
            pltpu.VMEM((2, S, STRIP), jnp.int32),
            pltpu.VMEM((2, CHUNK, DIM), jnp.float32),
            pltpu.SemaphoreType.DMA,
            pltpu.SemaphoreType.DMA,
            pltpu.SemaphoreType.DMA,
        ],
        compiler_params=pltpu.CompilerParams(use_tc_tiling_on_sc=False),
    )
    outf = run(x2d, weight)
    return outf.reshape(B0, B1, DIM)


# speedup vs baseline: 1.2434x; 1.2234x over previous
import jax
import jax.numpy as jnp
from jax import lax
from jax.experimental import pallas as pl
from jax.experimental.pallas import tpu as pltpu
from jax.experimental.pallas import tpu_sc as plsc

DIM = 64
PAD = 128
STRIP = 128
S = 2
NC = 2
NS = 16
NW = NC * NS
CHUNK = S * STRIP


def _emb_gather(x2d, table, out, idx_v, rows_v, gsem, wsem, isem):
    wid = lax.axis_index("s") * NC + lax.axis_index("c")
    n_strips = x2d.shape[0]
    per_w = n_strips // NW
    chunks = per_w // S
    pairs = chunks // 2
    base = wid * per_w

    def row0(c):
        return base + c * S

    def fire_gathers(b):
        for i in range(S):
            pltpu.async_copy(
                table.at[idx_v.at[b, i]],
                rows_v.at[b, pl.ds(i * STRIP, STRIP)],
                gsem,
            )

    def drain_gathers(b):
        for i in range(S):
            pltpu.make_async_copy(
                table.at[idx_v.at[b, i]],
                rows_v.at[b, pl.ds(i * STRIP, STRIP)],
                gsem,
            ).wait()

    def fire_writeback(c, b):
        pltpu.async_copy(
            rows_v.at[b],
            out.at[pl.ds(row0(c) * STRIP, CHUNK)],
            wsem,
        )

    def drain_writeback(c, b):
        pltpu.make_async_copy(
            rows_v.at[b],
            out.at[pl.ds(row0(c) * STRIP, CHUNK)],
            wsem,
        ).wait()

    def stage_idx(c, b):
        pltpu.async_copy(x2d.at[pl.ds(row0(c), S)], idx_v.at[b], isem)

    def drain_idx(c, b):
        pltpu.make_async_copy(
            x2d.at[pl.ds(row0(c), S)], idx_v.at[b], isem
        ).wait()

    stage_idx(0, 0)
    drain_idx(0, 0)
    stage_idx(1, 1)
    fire_gathers(0)

    def pair_body(g, carry):
        c0 = 2 * g

        @pl.when(g > 0)
        def _():
            drain_writeback(c0 - 1, 1)

        drain_idx(c0 + 1, 1)
        fire_gathers(1)
        drain_gathers(0)
        fire_writeback(c0, 0)

        @pl.when(g < pairs - 1)
        def _():
            stage_idx(c0 + 2, 0)

        c1 = 2 * g + 1
        drain_writeback(c1 - 1, 0)

        @pl.when(g < pairs - 1)
        def _():
            drain_idx(c1 + 1, 0)
            fire_gathers(0)

        drain_gathers(1)
        fire_writeback(c1, 1)

        @pl.when(g < pairs - 1)
        def _():
            stage_idx(c1 + 2, 1)

        return carry

    lax.fori_loop(0, pairs, pair_body, 0)
    drain_writeback(chunks - 1, 1)


def kernel(x, weight):
    B0, B1 = x.shape
    n = B0 * B1
    xf = x.reshape(n).astype(jnp.int32)
    n_strips = n // STRIP
    x2d = xf.reshape(n_strips, STRIP)
    wpad = jnp.pad(weight, ((0, 0), (0, PAD - DIM)))

    mesh = plsc.VectorSubcoreMesh(core_axis_name="c", subcore_axis_name="s")
    run = pl.kernel(
        _emb_gather,
        mesh=mesh,
        out_type=jax.ShapeDtypeStruct((n, PAD), jnp.float32),
        scratch_types=[
            pltpu.VMEM((2, S, STRIP), jnp.int32),
            pltpu.VMEM((2, CHUNK, PAD), jnp.float32),
            pltpu.SemaphoreType.DMA,
            pltpu.SemaphoreType.DMA,
            pltpu.SemaphoreType.DMA,
        ],
        compiler_params=pltpu.CompilerParams(use_tc_tiling_on_sc=False),
    )
    outf = run(x2d, wpad)
    return outf[:, :DIM].reshape(B0, B1, DIM)


# speedup vs baseline: 1.2823x; 1.0312x over previous
import jax
import jax.numpy as jnp
from jax import lax
from jax.experimental import pallas as pl
from jax.experimental.pallas import tpu as pltpu
from jax.experimental.pallas import tpu_sc as plsc

DIM = 64
PAD = 128
STRIP = 128
S = 2
NC = 2
NS = 16
NW = NC * NS
CHUNK = S * STRIP


def _emb_gather(x2d, table, out, idx_v, rows_v, gsem, wsem, isem):
    wid = lax.axis_index("s") * NC + lax.axis_index("c")
    n_strips = x2d.shape[0]
    per_w = n_strips // NW
    chunks = per_w // S
    pairs = chunks // 2
    base = wid * per_w

    def row0(c):
        return base + c * S

    def fire_gathers(b):
        for i in range(S):
            pltpu.async_copy(
                table.at[idx_v.at[b, i]],
                rows_v.at[b, pl.ds(i * STRIP, STRIP)],
                gsem,
            )

    def drain_gathers(b):
        for i in range(S):
            pltpu.make_async_copy(
                table.at[idx_v.at[b, i]],
                rows_v.at[b, pl.ds(i * STRIP, STRIP)],
                gsem,
            ).wait()

    def fire_writeback(c, b):
        pltpu.async_copy(
            rows_v.at[b, :, pl.ds(0, DIM)],
            out.at[pl.ds(row0(c) * STRIP, CHUNK), pl.ds(0, DIM)],
            wsem,
        )

    def drain_writeback(c, b):
        pltpu.make_async_copy(
            rows_v.at[b, :, pl.ds(0, DIM)],
            out.at[pl.ds(row0(c) * STRIP, CHUNK), pl.ds(0, DIM)],
            wsem,
        ).wait()

    def stage_idx(c, b):
        pltpu.async_copy(x2d.at[pl.ds(row0(c), S)], idx_v.at[b], isem)

    def drain_idx(c, b):
        pltpu.make_async_copy(
            x2d.at[pl.ds(row0(c), S)], idx_v.at[b], isem
        ).wait()

    stage_idx(0, 0)
    drain_idx(0, 0)
    stage_idx(1, 1)
    fire_gathers(0)

    def pair_body(g, carry):
        c0 = 2 * g

        @pl.when(g > 0)
        def _():
            drain_writeback(c0 - 1, 1)

        drain_idx(c0 + 1, 1)
        fire_gathers(1)
        drain_gathers(0)
        fire_writeback(c0, 0)

        @pl.when(g < pairs - 1)
        def _():
            stage_idx(c0 + 2, 0)

        c1 = 2 * g + 1
        drain_writeback(c1 - 1, 0)

        @pl.when(g < pairs - 1)
        def _():
            drain_idx(c1 + 1, 0)
            fire_gathers(0)

        drain_gathers(1)
        fire_writeback(c1, 1)

        @pl.when(g < pairs - 1)
        def _():
            stage_idx(c1 + 2, 1)

        return carry

    lax.fori_loop(0, pairs, pair_body, 0)
    drain_writeback(chunks - 1, 1)


def kernel(x, weight):
    B0, B1 = x.shape
    n = B0 * B1
    xf = x.reshape(n).astype(jnp.int32)
    n_strips = n // STRIP
    x2d = xf.reshape(n_strips, STRIP)
    wpad = jnp.pad(weight, ((0, 0), (0, PAD - DIM)))

    mesh = plsc.VectorSubcoreMesh(core_axis_name="c", subcore_axis_name="s")
    run = pl.kernel(
        _emb_gather,
        mesh=mesh,
        out_type=jax.ShapeDtypeStruct((n, PAD), jnp.float32),
        scratch_types=[
            pltpu.VMEM((2, S, STRIP), jnp.int32),
            pltpu.VMEM((2, CHUNK, PAD), jnp.float32),
            pltpu.SemaphoreType.DMA,
            pltpu.SemaphoreType.DMA,
            pltpu.SemaphoreType.DMA,
        ],
        compiler_params=pltpu.CompilerParams(use_tc_tiling_on_sc=False),
    )
    outf = run(x2d, wpad)
    return outf[:, :DIM].reshape(B0, B1, DIM)


# speedup vs baseline: 1.3499x; 1.0528x over previous
import jax
import jax.numpy as jnp
from jax import lax
from jax.experimental import pallas as pl
from jax.experimental.pallas import tpu as pltpu
from jax.experimental.pallas import tpu_sc as plsc

DIM = 64
PAD = 128
STRIP = 128
S = 2
NC = 2
NS = 16
NW = NC * NS
CHUNK = S * STRIP


def _emb_gather(x2d, table, out, idx_v, rows_v, gsem, wsem, isem):
    wid = lax.axis_index("s") * NC + lax.axis_index("c")
    n_strips = x2d.shape[0]
    per_w = n_strips // NW
    chunks = per_w // S
    pairs = chunks // 2
    base = wid * per_w

    def row0(c):
        return base + c * S

    def fire_gathers(b):
        for i in range(S):
            pltpu.async_copy(
                table.at[idx_v.at[b, i]],
                rows_v.at[b, pl.ds(i * STRIP, STRIP)],
                gsem,
            )

    def drain_gathers(b):
        for i in range(S):
            pltpu.make_async_copy(
                table.at[idx_v.at[b, i]],
                rows_v.at[b, pl.ds(i * STRIP, STRIP)],
                gsem,
            ).wait()

    def fire_writeback(c, b):
        pltpu.async_copy(
            rows_v.at[b],
            out.at[pl.ds(row0(c) * STRIP, CHUNK), pl.ds(0, DIM)],
            wsem,
        )

    def drain_writeback(c, b):
        pltpu.make_async_copy(
            rows_v.at[b],
            out.at[pl.ds(row0(c) * STRIP, CHUNK), pl.ds(0, DIM)],
            wsem,
        ).wait()

    def stage_idx(c, b):
        pltpu.async_copy(x2d.at[pl.ds(row0(c), S)], idx_v.at[b], isem)

    def drain_idx(c, b):
        pltpu.make_async_copy(
            x2d.at[pl.ds(row0(c), S)], idx_v.at[b], isem
        ).wait()

    stage_idx(0, 0)
    drain_idx(0, 0)
    stage_idx(1, 1)
    fire_gathers(0)

    def pair_body(g, carry):
        c0 = 2 * g

        @pl.when(g > 0)
        def _():
            drain_writeback(c0 - 1, 1)

        drain_idx(c0 + 1, 1)
        fire_gathers(1)
        drain_gathers(0)
        fire_writeback(c0, 0)

        @pl.when(g < pairs - 1)
        def _():
            stage_idx(c0 + 2, 0)

        c1 = 2 * g + 1
        drain_writeback(c1 - 1, 0)

        @pl.when(g < pairs - 1)
        def _():
            drain_idx(c1 + 1, 0)
            fire_gathers(0)

        drain_gathers(1)
        fire_writeback(c1, 1)

        @pl.when(g < pairs - 1)
        def _():
            stage_idx(c1 + 2, 1)

        return carry

    lax.fori_loop(0, pairs, pair_body, 0)
    drain_writeback(chunks - 1, 1)


def kernel(x, weight):
    B0, B1 = x.shape
    n = B0 * B1
    xf = x.reshape(n).astype(jnp.int32)
    n_strips = n // STRIP
    x2d = xf.reshape(n_strips, STRIP)

    mesh = plsc.VectorSubcoreMesh(core_axis_name="c", subcore_axis_name="s")
    run = pl.kernel(
        _emb_gather,
        mesh=mesh,
        out_type=jax.ShapeDtypeStruct((n, PAD), jnp.float32),
        scratch_types=[
            pltpu.VMEM((2, S, STRIP), jnp.int32),
            pltpu.VMEM((2, CHUNK, DIM), jnp.float32),
            pltpu.SemaphoreType.DMA,
            pltpu.SemaphoreType.DMA,
            pltpu.SemaphoreType.DMA,
        ],
        compiler_params=pltpu.CompilerParams(use_tc_tiling_on_sc=False),
    )
    outf = run(x2d, weight)
    return outf[:, :DIM].reshape(B0, B1, DIM)
